# Initial kernel scaffold; baseline (speedup 1.0000x reference)
import jax
import jax.numpy as jnp
from jax.experimental import pallas as pl


def kernel(x, edge_index, Wl1, Wr1, a1, b1, Wl2, Wr2, a2, b2, Wlin, blin):
    raise NotImplementedError("write your pallas kernel here")


# speedup vs baseline: 3.0381x; 3.0381x over previous
import functools

import jax
import jax.numpy as jnp
from jax import lax
from jax.experimental import pallas as pl
from jax.experimental.pallas import tpu as pltpu
from jax.experimental.pallas import tpu_sc as plsc

_N = 10000
_E = 320000
_P = 16
_HALF = 5000
_HPAD = 5120
_NPAD = 10016
_NC = 2
_NS = 16
_U = 8


def _mesh():
    return plsc.VectorSubcoreMesh(core_axis_name="c", subcore_axis_name="s")


def _mm_body(x_ref, w_ref, bi_ref, bo_ref, o_ref, *, elu_in):
    xb = x_ref[...] + bi_ref[...]
    if elu_in:
        xb = jnp.where(xb > 0.0, xb, jnp.exp(xb) - 1.0)
    o_ref[...] = (
        jnp.dot(xb, w_ref[...], preferred_element_type=jnp.float32) + bo_ref[...]
    )


def _mm(x, w, bi, bo, elu_in):
    m, k = x.shape
    cols = w.shape[1]
    bm = 256
    return pl.pallas_call(
        functools.partial(_mm_body, elu_in=elu_in),
        grid=(m // bm,),
        in_specs=[
            pl.BlockSpec((bm, k), lambda i: (i, 0)),
            pl.BlockSpec((k, cols), lambda i: (0, 0)),
            pl.BlockSpec((1, k), lambda i: (0, 0)),
            pl.BlockSpec((1, cols), lambda i: (0, 0)),
        ],
        out_specs=pl.BlockSpec((bm, cols), lambda i: (i, 0)),
        out_shape=jax.ShapeDtypeStruct((m, cols), jnp.float32),
    )(x, w, bi.reshape(1, k), bo.reshape(1, cols))


def _logits_call(xl, xr, src, dst, att, h_heads, ch):
    hd = h_heads * ch
    ept = _E // (_NC * _NS)
    c = 80
    nch = ept // c
    zr = 624
    tail = _N - _NS * zr

    @functools.partial(
        pl.kernel,
        mesh=_mesh(),
        compiler_params=pltpu.CompilerParams(
            needs_layout_passes=False, use_tc_tiling_on_sc=False),
        out_type=[
            jax.ShapeDtypeStruct((_E, _P), jnp.float32),
            jax.ShapeDtypeStruct((_NC, _NPAD, _P), jnp.float32),
        ],
        scratch_types=[
            pltpu.VMEM((c,), jnp.int32),
            pltpu.VMEM((c,), jnp.int32),
            pltpu.VMEM((c, hd), jnp.float32),
            pltpu.VMEM((c, hd), jnp.float32),
            pltpu.VMEM((c, _P), jnp.float32),
            pltpu.VMEM((hd,), jnp.float32),
            pltpu.VMEM((zr, _P), jnp.float32),
            pltpu.VMEM_SHARED((_N, _P), jnp.float32),
            pltpu.SemaphoreType.DMA,
            pltpu.SemaphoreType.DMA,
        ],
    )
    def k(xl_h, xr_h, src_h, dst_h, att_h, p_h, den_h,
          src_v, dst_v, xl_v, xr_v, p_v, att_v, zb_v, den_s, s0, s1):
        cid = lax.axis_index("c")
        sid = lax.axis_index("s")
        wid = sid * _NC + cid
        i16 = lax.iota(jnp.int32, 16)
        zf = jnp.zeros((16,), jnp.float32)

        pltpu.sync_copy(att_h, att_v)

        def zrow(i, carry):
            zb_v[i, :] = zf
            return carry

        lax.fori_loop(0, zr, zrow, None)

        def zprow(i, carry):
            p_v[i, :] = zf
            return carry

        lax.fori_loop(0, c, zprow, None)
        pltpu.sync_copy(zb_v, den_s.at[pl.ds(sid * zr, zr)])

        @pl.when(sid == _NS - 1)
        def _ztail():
            pltpu.sync_copy(zb_v.at[pl.ds(0, tail)],
                            den_s.at[pl.ds(_NS * zr, tail)])

        plsc.subcore_barrier()

        def chunk(ci, carry):
            e0 = wid * ept + ci * c
            pltpu.sync_copy(src_h.at[pl.ds(e0, c)], src_v)
            pltpu.sync_copy(dst_h.at[pl.ds(e0, c)], dst_v)
            ca = pltpu.async_copy(xl_h.at[src_v], xl_v, s0)
            cb = pltpu.async_copy(xr_h.at[dst_v], xr_v, s1)
            ca.wait()
            cb.wait()

            def group(g, carry2):
                rv = g * 16 + i16
                for h in range(h_heads):
                    def chf(c0, acc):
                        for u in range(_U):
                            chn = h * ch + c0 * _U + u
                            chv = jnp.full((16,), chn, jnp.int32)
                            a = plsc.load_gather(xl_v, [rv, chv])
                            b = plsc.load_gather(xr_v, [rv, chv])
                            av = plsc.load_gather(att_v, [chv])
                            z = a + b
                            z = jnp.where(z >= 0.0, z, 0.2 * z)
                            acc = acc + z * av
                        return acc

                    acc = lax.fori_loop(0, ch // _U, chf, zf)
                    plsc.store_scatter(
                        p_v, [rv, jnp.full((16,), h, jnp.int32)], jnp.exp(acc))
                return carry2

            lax.fori_loop(0, c // 16, group, None)
            pltpu.sync_copy(p_v, p_h.at[pl.ds(e0, c)])
            pltpu.sync_copy(p_v, den_s.at[dst_v], add=True)
            return carry

        lax.fori_loop(0, nch, chunk, None)
        plsc.subcore_barrier()
        r0 = sid * zr
        pltpu.sync_copy(den_s.at[pl.ds(r0, zr)], den_h.at[cid, pl.ds(r0, zr)])

        @pl.when(sid == _NS - 1)
        def _wtail():
            pltpu.sync_copy(den_s.at[pl.ds(_NS * zr, tail)],
                            den_h.at[cid, pl.ds(_NS * zr, tail)])

    return k(xl, xr, src, dst, att)


def _aggregate_call(xl, src, dst, p, den, h_heads, ch):
    hd = h_heads * ch
    ept = _E // _NS
    c = 80
    nch = ept // c
    rpt = _HPAD // _NS
    vr = hd // 16

    @functools.partial(
        pl.kernel,
        mesh=_mesh(),
        compiler_params=pltpu.CompilerParams(
            needs_layout_passes=False, use_tc_tiling_on_sc=False),
        out_type=jax.ShapeDtypeStruct((_NC, _HPAD, hd), jnp.float32),
        scratch_types=[
            pltpu.VMEM((c,), jnp.int32),
            pltpu.VMEM((c,), jnp.int32),
            pltpu.VMEM((c,), jnp.int32),
            pltpu.VMEM((c, hd), jnp.float32),
            pltpu.VMEM((c, _P), jnp.float32),
            pltpu.VMEM((16, hd), jnp.float32),
            pltpu.VMEM((16, _P), jnp.float32),
            pltpu.VMEM((16, _P), jnp.float32),
            pltpu.VMEM_SHARED((_HPAD, hd), jnp.float32),
            pltpu.SemaphoreType.DMA,
        ],
    )
    def k(xl_h, src_h, dst_h, p_h, den_h, out_h,
          src_v, dst_v, loc_v, xw_v, p_v, ow_v, d0_v, d1_v, acc_s, s0):
        cid = lax.axis_index("c")
        sid = lax.axis_index("s")
        i16 = lax.iota(jnp.int32, 16)
        zf = jnp.zeros((16,), jnp.float32)

        def zrow(i, carry):
            xw_v[i // vr, pl.ds((i % vr) * 16, 16)] = zf
            return carry

        lax.fori_loop(0, c * vr, zrow, None)
        for j in range(rpt // c):
            pltpu.sync_copy(xw_v, acc_s.at[pl.ds(sid * rpt + j * c, c)])
        plsc.subcore_barrier()

        base = cid * _HALF

        def chunk(ci, carry):
            e0 = sid * ept + ci * c
            pltpu.sync_copy(src_h.at[pl.ds(e0, c)], src_v)
            pltpu.sync_copy(dst_h.at[pl.ds(e0, c)], dst_v)
            pltpu.sync_copy(p_h.at[pl.ds(e0, c)], p_v)
            pltpu.async_copy(xl_h.at[src_v], xw_v, s0).wait()

            def lgrp(g, carry2):
                dv = dst_v[pl.ds(g * 16, 16)]
                lv = dv - base
                ok = (lv >= 0) & (lv < _HALF)
                loc_v[pl.ds(g * 16, 16)] = jnp.where(ok, lv, _HALF)
                return carry2

            lax.fori_loop(0, c // 16, lgrp, None)

            def group(g, carry2):
                rv = g * 16 + i16
                for h in range(h_heads):
                    wv = plsc.load_gather(
                        p_v, [rv, jnp.full((16,), h, jnp.int32)])

                    def chf(c0, carry3):
                        for u in range(_U):
                            chn = h * ch + c0 * _U + u
                            chv = jnp.full((16,), chn, jnp.int32)
                            a = plsc.load_gather(xw_v, [rv, chv])
                            plsc.store_scatter(xw_v, [rv, chv], a * wv)
                        return carry3

                    lax.fori_loop(0, ch // _U, chf, None)
                return carry2

            lax.fori_loop(0, c // 16, group, None)
            pltpu.sync_copy(xw_v, acc_s.at[loc_v], add=True)
            return carry

        lax.fori_loop(0, nch, chunk, None)
        plsc.subcore_barrier()

        def wgrp(g, carry):
            r0 = sid * rpt + g * 16

            @pl.when(r0 < _HALF)
            def _w():
                pltpu.sync_copy(acc_s.at[pl.ds(r0, 16)], xw_v.at[pl.ds(0, 16)])
                gb = base + r0
                pltpu.sync_copy(den_h.at[0, pl.ds(gb, 16)], d0_v)
                pltpu.sync_copy(den_h.at[1, pl.ds(gb, 16)], d1_v)
                for h in range(h_heads):
                    hv = jnp.full((16,), h, jnp.int32)
                    dv = (plsc.load_gather(d0_v, [i16, hv])
                          + plsc.load_gather(d1_v, [i16, hv]))
                    inv = 1.0 / (dv + 1e-16)

                    def chf(c0, carry3):
                        for u in range(_U):
                            chn = h * ch + c0 * _U + u
                            chv = jnp.full((16,), chn, jnp.int32)
                            a = plsc.load_gather(xw_v, [i16, chv])
                            plsc.store_scatter(ow_v, [i16, chv], a * inv)
                        return carry3

                    lax.fori_loop(0, ch // _U, chf, None)
                pltpu.sync_copy(ow_v, out_h.at[cid, pl.ds(r0, 16)])
            return carry

        lax.fori_loop(0, rpt // 16, wgrp, None)

    return k(xl, src, dst, p, den)


def _gat_layer(xlr, src, dst, att, h_heads, ch):
    hd = h_heads * ch
    xl = xlr[:, :hd]
    xr = xlr[:, hd:]
    p, den = _logits_call(xl, xr, src, dst, att.reshape(-1), h_heads, ch)
    halves = _aggregate_call(xl, src, dst, p, den, h_heads, ch)
    return jnp.concatenate([halves[0, :_HALF], halves[1, :_HALF]], axis=0)


def kernel(x, edge_index, Wl1, Wr1, a1, b1, Wl2, Wr2, a2, b2, Wlin, blin):
    src = edge_index[0]
    dst = edge_index[1]
    d = x.shape[1]
    pad = 10240 - _N

    xp = jnp.pad(x, ((0, pad), (0, 0)))
    lr1 = _mm(xp, jnp.concatenate([Wl1, Wr1], axis=1),
              jnp.zeros((d,), jnp.float32), jnp.zeros((512,), jnp.float32),
              False)[:_N]
    h1 = _gat_layer(lr1, src, dst, a1, 8, 32)

    h1p = jnp.pad(h1, ((0, pad), (0, 0)))
    lr2 = _mm(h1p, jnp.concatenate([Wl2, Wr2], axis=1),
              b1, jnp.zeros((256,), jnp.float32), True)[:_N]
    h2 = _gat_layer(lr2, src, dst, a2, 1, 128)

    h2p = jnp.pad(h2, ((0, pad), (0, 0)))
    out = _mm(h2p, Wlin, b2, blin, True)[:_N]
    return out


# speedup vs baseline: 3.0601x; 1.0072x over previous
import functools

import jax
import jax.numpy as jnp
from jax import lax
from jax.experimental import pallas as pl
from jax.experimental.pallas import tpu as pltpu
from jax.experimental.pallas import tpu_sc as plsc

_N = 10000
_E = 320000
_P = 16
_HALF = 5000
_HPAD = 5120
_NPAD = 10016
_NC = 2
_NS = 16
_U = 8


def _mesh():
    return plsc.VectorSubcoreMesh(core_axis_name="c", subcore_axis_name="s")


def _mm_body(x_ref, w_ref, bi_ref, bo_ref, o_ref, *, elu_in):
    xb = x_ref[...] + bi_ref[...]
    if elu_in:
        xb = jnp.where(xb > 0.0, xb, jnp.exp(xb) - 1.0)
    o_ref[...] = (
        jnp.dot(xb, w_ref[...], preferred_element_type=jnp.float32) + bo_ref[...]
    )


def _mm(x, w, bi, bo, elu_in):
    m, k = x.shape
    cols = w.shape[1]
    bm = 256
    return pl.pallas_call(
        functools.partial(_mm_body, elu_in=elu_in),
        grid=(m // bm,),
        in_specs=[
            pl.BlockSpec((bm, k), lambda i: (i, 0)),
            pl.BlockSpec((k, cols), lambda i: (0, 0)),
            pl.BlockSpec((1, k), lambda i: (0, 0)),
            pl.BlockSpec((1, cols), lambda i: (0, 0)),
        ],
        out_specs=pl.BlockSpec((bm, cols), lambda i: (i, 0)),
        out_shape=jax.ShapeDtypeStruct((m, cols), jnp.float32),
    )(x, w, bi.reshape(1, k), bo.reshape(1, cols))


def _logits_call(xl, xr, src, dst, att, h_heads, ch):
    hd = h_heads * ch
    ept = _E // (_NC * _NS)
    c = 80
    nch = ept // c
    zr = 624
    tail = _N - _NS * zr

    @functools.partial(
        pl.kernel,
        mesh=_mesh(),
        compiler_params=pltpu.CompilerParams(
            needs_layout_passes=False, use_tc_tiling_on_sc=False),
        out_type=[
            jax.ShapeDtypeStruct((_E, _P), jnp.float32),
            jax.ShapeDtypeStruct((_NC, _NPAD, _P), jnp.float32),
        ],
        scratch_types=[
            pltpu.VMEM((c,), jnp.int32),
            pltpu.VMEM((c,), jnp.int32),
            pltpu.VMEM((c, hd), jnp.float32),
            pltpu.VMEM((c, hd), jnp.float32),
            pltpu.VMEM((c, _P), jnp.float32),
            pltpu.VMEM((hd,), jnp.float32),
            pltpu.VMEM((zr, _P), jnp.float32),
            pltpu.VMEM_SHARED((_N, _P), jnp.float32),
            pltpu.SemaphoreType.DMA,
            pltpu.SemaphoreType.DMA,
        ],
    )
    def k(xl_h, xr_h, src_h, dst_h, att_h, p_h, den_h,
          src_v, dst_v, xl_v, xr_v, p_v, att_v, zb_v, den_s, s0, s1):
        cid = lax.axis_index("c")
        sid = lax.axis_index("s")
        wid = sid * _NC + cid
        i16 = lax.iota(jnp.int32, 16)
        zf = jnp.zeros((16,), jnp.float32)

        pltpu.sync_copy(att_h, att_v)

        def zrow(i, carry):
            zb_v[i, :] = zf
            return carry

        lax.fori_loop(0, zr, zrow, None)

        def zprow(i, carry):
            p_v[i, :] = zf
            return carry

        lax.fori_loop(0, c, zprow, None)
        pltpu.sync_copy(zb_v, den_s.at[pl.ds(sid * zr, zr)])

        @pl.when(sid == _NS - 1)
        def _ztail():
            pltpu.sync_copy(zb_v.at[pl.ds(0, tail)],
                            den_s.at[pl.ds(_NS * zr, tail)])

        plsc.subcore_barrier()

        def chunk(ci, carry):
            e0 = wid * ept + ci * c
            pltpu.sync_copy(src_h.at[pl.ds(e0, c)], src_v)
            pltpu.sync_copy(dst_h.at[pl.ds(e0, c)], dst_v)
            ca = pltpu.async_copy(xl_h.at[src_v], xl_v, s0)
            cb = pltpu.async_copy(xr_h.at[dst_v], xr_v, s1)
            ca.wait()
            cb.wait()

            def group(g, carry2):
                rv = g * 16 + i16
                for h in range(h_heads):
                    def chf(c0, accs):
                        accs = list(accs)
                        for u in range(_U):
                            chn = h * ch + c0 * _U + u
                            chv = jnp.full((16,), chn, jnp.int32)
                            a = plsc.load_gather(xl_v, [rv, chv])
                            b = plsc.load_gather(xr_v, [rv, chv])
                            av = plsc.load_gather(att_v, [chv])
                            z = a + b
                            z = jnp.where(z >= 0.0, z, 0.2 * z)
                            accs[u % 4] = accs[u % 4] + z * av
                        return tuple(accs)

                    a0, a1, a2, a3 = lax.fori_loop(
                        0, ch // _U, chf, (zf, zf, zf, zf))
                    acc = (a0 + a1) + (a2 + a3)
                    plsc.store_scatter(
                        p_v, [rv, jnp.full((16,), h, jnp.int32)], jnp.exp(acc))
                return carry2

            lax.fori_loop(0, c // 16, group, None)
            pltpu.sync_copy(p_v, p_h.at[pl.ds(e0, c)])
            pltpu.sync_copy(p_v, den_s.at[dst_v], add=True)
            return carry

        lax.fori_loop(0, nch, chunk, None)
        plsc.subcore_barrier()
        r0 = sid * zr
        pltpu.sync_copy(den_s.at[pl.ds(r0, zr)], den_h.at[cid, pl.ds(r0, zr)])

        @pl.when(sid == _NS - 1)
        def _wtail():
            pltpu.sync_copy(den_s.at[pl.ds(_NS * zr, tail)],
                            den_h.at[cid, pl.ds(_NS * zr, tail)])

    return k(xl, xr, src, dst, att)


def _aggregate_call(xl, src, dst, p, den, h_heads, ch):
    hd = h_heads * ch
    ept = _E // _NS
    c = 80
    nch = ept // c
    rpt = _HPAD // _NS
    vr = hd // 16

    @functools.partial(
        pl.kernel,
        mesh=_mesh(),
        compiler_params=pltpu.CompilerParams(
            needs_layout_passes=False, use_tc_tiling_on_sc=False),
        out_type=jax.ShapeDtypeStruct((_NC, _HPAD, hd), jnp.float32),
        scratch_types=[
            pltpu.VMEM((c,), jnp.int32),
            pltpu.VMEM((c,), jnp.int32),
            pltpu.VMEM((c,), jnp.int32),
            pltpu.VMEM((c, hd), jnp.float32),
            pltpu.VMEM((c, hd), jnp.float32),
            pltpu.VMEM((c, _P), jnp.float32),
            pltpu.VMEM((16, hd), jnp.float32),
            pltpu.VMEM((16, _P), jnp.float32),
            pltpu.VMEM((16, _P), jnp.float32),
            pltpu.VMEM_SHARED((_HPAD, hd), jnp.float32),
            pltpu.SemaphoreType.DMA,
        ],
    )
    def k(xl_h, src_h, dst_h, p_h, den_h, out_h,
          src_v, dst_v, loc_v, xw_v, sc_v, p_v, ow_v, d0_v, d1_v, acc_s, s0):
        cid = lax.axis_index("c")
        sid = lax.axis_index("s")
        i16 = lax.iota(jnp.int32, 16)
        zf = jnp.zeros((16,), jnp.float32)

        def zrow(i, carry):
            xw_v[i // vr, pl.ds((i % vr) * 16, 16)] = zf
            return carry

        lax.fori_loop(0, c * vr, zrow, None)
        for j in range(rpt // c):
            pltpu.sync_copy(xw_v, acc_s.at[pl.ds(sid * rpt + j * c, c)])
        plsc.subcore_barrier()

        base = cid * _HALF

        def chunk(ci, carry):
            e0 = sid * ept + ci * c
            pltpu.sync_copy(src_h.at[pl.ds(e0, c)], src_v)
            pltpu.sync_copy(dst_h.at[pl.ds(e0, c)], dst_v)
            pltpu.sync_copy(p_h.at[pl.ds(e0, c)], p_v)
            pltpu.async_copy(xl_h.at[src_v], xw_v, s0).wait()

            def lgrp(g, carry2):
                dv = dst_v[pl.ds(g * 16, 16)]
                lv = dv - base
                ok = (lv >= 0) & (lv < _HALF)
                loc_v[pl.ds(g * 16, 16)] = jnp.where(ok, lv, _HALF)
                return carry2

            lax.fori_loop(0, c // 16, lgrp, None)

            def group(g, carry2):
                rv = g * 16 + i16
                for h in range(h_heads):
                    wv = plsc.load_gather(
                        p_v, [rv, jnp.full((16,), h, jnp.int32)])

                    def chf(c0, carry3):
                        for u in range(_U):
                            chn = h * ch + c0 * _U + u
                            chv = jnp.full((16,), chn, jnp.int32)
                            a = plsc.load_gather(xw_v, [rv, chv])
                            plsc.store_scatter(sc_v, [rv, chv], a * wv)
                        return carry3

                    lax.fori_loop(0, ch // _U, chf, None)
                return carry2

            lax.fori_loop(0, c // 16, group, None)
            pltpu.sync_copy(sc_v, acc_s.at[loc_v], add=True)
            return carry

        lax.fori_loop(0, nch, chunk, None)
        plsc.subcore_barrier()

        def wgrp(g, carry):
            r0 = sid * rpt + g * 16

            @pl.when(r0 < _HALF)
            def _w():
                pltpu.sync_copy(acc_s.at[pl.ds(r0, 16)], xw_v.at[pl.ds(0, 16)])
                gb = base + r0
                pltpu.sync_copy(den_h.at[0, pl.ds(gb, 16)], d0_v)
                pltpu.sync_copy(den_h.at[1, pl.ds(gb, 16)], d1_v)
                for h in range(h_heads):
                    hv = jnp.full((16,), h, jnp.int32)
                    dv = (plsc.load_gather(d0_v, [i16, hv])
                          + plsc.load_gather(d1_v, [i16, hv]))
                    inv = 1.0 / (dv + 1e-16)

                    def chf(c0, carry3):
                        for u in range(_U):
                            chn = h * ch + c0 * _U + u
                            chv = jnp.full((16,), chn, jnp.int32)
                            a = plsc.load_gather(xw_v, [i16, chv])
                            plsc.store_scatter(ow_v, [i16, chv], a * inv)
                        return carry3

                    lax.fori_loop(0, ch // _U, chf, None)
                pltpu.sync_copy(ow_v, out_h.at[cid, pl.ds(r0, 16)])
            return carry

        lax.fori_loop(0, rpt // 16, wgrp, None)

    return k(xl, src, dst, p, den)


def _gat_layer(xlr, src, dst, att, h_heads, ch):
    hd = h_heads * ch
    xl = xlr[:, :hd]
    xr = xlr[:, hd:]
    p, den = _logits_call(xl, xr, src, dst, att.reshape(-1), h_heads, ch)
    halves = _aggregate_call(xl, src, dst, p, den, h_heads, ch)
    return jnp.concatenate([halves[0, :_HALF], halves[1, :_HALF]], axis=0)


def kernel(x, edge_index, Wl1, Wr1, a1, b1, Wl2, Wr2, a2, b2, Wlin, blin):
    src = edge_index[0]
    dst = edge_index[1]
    d = x.shape[1]
    pad = 10240 - _N

    xp = jnp.pad(x, ((0, pad), (0, 0)))
    lr1 = _mm(xp, jnp.concatenate([Wl1, Wr1], axis=1),
              jnp.zeros((d,), jnp.float32), jnp.zeros((512,), jnp.float32),
              False)[:_N]
    h1 = _gat_layer(lr1, src, dst, a1, 8, 32)

    h1p = jnp.pad(h1, ((0, pad), (0, 0)))
    lr2 = _mm(h1p, jnp.concatenate([Wl2, Wr2], axis=1),
              b1, jnp.zeros((256,), jnp.float32), True)[:_N]
    h2 = _gat_layer(lr2, src, dst, a2, 1, 128)

    h2p = jnp.pad(h2, ((0, pad), (0, 0)))
    out = _mm(h2p, Wlin, b2, blin, True)[:_N]
    return out
